# baseline (speedup 1.0000x reference)
import jax
import jax.numpy as jnp
from jax.experimental import pallas as pl

_BS = 128


def _add_body(x_ref, pe_ref, o_ref):
    o_ref[...] = x_ref[...] + pe_ref[...][:, None, :]


def kernel(x, pe):
    S, B, D = x.shape
    pe = pe[:S]
    bs = _BS if S % _BS == 0 else pl.cdiv(S, pl.cdiv(S, _BS))
    if S % bs != 0:
        bs = S
    return pl.pallas_call(
        _add_body,
        grid=(S // bs,),
        in_specs=[
            pl.BlockSpec((bs, B, D), lambda i: (i, 0, 0)),
            pl.BlockSpec((bs, D), lambda i: (i, 0)),
        ],
        out_specs=pl.BlockSpec((bs, B, D), lambda i: (i, 0, 0)),
        out_shape=jax.ShapeDtypeStruct((S, B, D), x.dtype),
    )(x, pe)


# speedup vs baseline: 1.0977x; 1.0977x over previous
import jax
import jax.numpy as jnp
from jax.experimental import pallas as pl

_BS = 256


def _add_body(x_ref, pe_ref, o_ref):
    o_ref[...] = x_ref[...] + pe_ref[...][:, None, :]


def kernel(x, pe):
    S, B, D = x.shape
    pe = pe[:S]
    bs = _BS if S % _BS == 0 else pl.cdiv(S, pl.cdiv(S, _BS))
    if S % bs != 0:
        bs = S
    return pl.pallas_call(
        _add_body,
        grid=(S // bs,),
        in_specs=[
            pl.BlockSpec((bs, B, D), lambda i: (i, 0, 0)),
            pl.BlockSpec((bs, D), lambda i: (i, 0)),
        ],
        out_specs=pl.BlockSpec((bs, B, D), lambda i: (i, 0, 0)),
        out_shape=jax.ShapeDtypeStruct((S, B, D), x.dtype),
    )(x, pe)
